# baseline (speedup 1.0000x reference)
import jax
import jax.numpy as jnp
from jax import lax
from jax.experimental import pallas as pl
from jax.experimental.pallas import tpu as pltpu
from jax.experimental.pallas import tpu_sc as plsc

M = 4096
NC, NS, L = 2, 16, 16
NW = NC * NS
NBT = M // 8
BANDS_PER_W = NBT // NW


def _body(x_hbm, out_hbm, x_v, y_v, band0, band1, sem0, sem1):
    c = lax.axis_index("c")
    s = lax.axis_index("s")
    wid = s * NC + c

    pltpu.sync_copy(x_hbm, x_v)

    @plsc.parallel_loop(0, M, L)
    def build(tt):
        v = x_v[pl.ds(tt, L)]
        t = tt + lax.iota(jnp.int32, L)
        plsc.store_scatter(y_v, [(M - 1) + t], v)
        plsc.store_scatter(y_v, [(M - 1) - t], v)

    HALF = M // 2

    def fill(band, it, h):
        base = (M - 1) - 8 * it

        @plsc.parallel_loop(h * HALF, (h + 1) * HALF, L, unroll=4)
        def chunk(j):
            for sub in range(8):
                band[sub, pl.ds(j, L)] = y_v[pl.ds((base - sub) + j, L)]

    it0 = wid * BANDS_PER_W

    def emit(band, it, sem):
        for h in range(2):
            fill(band, it, h)
            pltpu.async_copy(
                band.at[:, pl.ds(h * HALF, HALF)],
                out_hbm.at[pl.ds(8 * it, 8), pl.ds(h * HALF, HALF)],
                sem,
            )

    def drain(band, sem):
        pltpu.make_async_copy(band, out_hbm.at[pl.ds(0, 8), :], sem).wait()

    emit(band0, it0, sem0)

    def bands(k, _):
        it = it0 + 2 * k + 1
        emit(band1, it, sem1)
        drain(band0, sem0)
        emit(band0, it + 1, sem0)
        drain(band1, sem1)
        return 0

    lax.fori_loop(0, (BANDS_PER_W - 2) // 2, bands, 0)

    emit(band1, it0 + BANDS_PER_W - 1, sem1)
    drain(band0, sem0)
    drain(band1, sem1)


_mesh = plsc.VectorSubcoreMesh(core_axis_name="c", subcore_axis_name="s")

_toeplitz = pl.kernel(
    _body,
    out_type=jax.ShapeDtypeStruct((M, M), jnp.float32),
    mesh=_mesh,
    compiler_params=pltpu.CompilerParams(
        needs_layout_passes=False, use_tc_tiling_on_sc=True
    ),
    scratch_types=[
        pltpu.VMEM((M,), jnp.float32),
        pltpu.VMEM((2 * M,), jnp.float32),
        pltpu.VMEM((8, M), jnp.float32),
        pltpu.VMEM((8, M), jnp.float32),
        pltpu.SemaphoreType.DMA,
        pltpu.SemaphoreType.DMA,
    ],
)


@jax.jit
def kernel(input):
    out = _toeplitz(input.reshape(M).astype(jnp.float32))
    return out[None, :, :]


# speedup vs baseline: 1.0224x; 1.0224x over previous
import jax
import jax.numpy as jnp
from jax import lax
from jax.experimental import pallas as pl
from jax.experimental.pallas import tpu as pltpu
from jax.experimental.pallas import tpu_sc as plsc

M = 4096
NC, NS, L = 2, 16, 16
NW = NC * NS
NBT = M // 8
BANDS_PER_W = NBT // NW


def _body(x_hbm, out_hbm, x_v, y_v, band0, band1, sem0, sem1):
    c = lax.axis_index("c")
    s = lax.axis_index("s")
    wid = s * NC + c

    pltpu.sync_copy(x_hbm, x_v)

    @plsc.parallel_loop(0, M, L)
    def build(tt):
        v = x_v[pl.ds(tt, L)]
        t = tt + lax.iota(jnp.int32, L)
        plsc.store_scatter(y_v, [(M - 1) + t], v)
        plsc.store_scatter(y_v, [(M - 1) - t], v)

    def fill(band, it):
        base = (M - 1) - 8 * it

        @plsc.parallel_loop(0, M, L, unroll=4)
        def chunk(j):
            for sub in range(8):
                band[sub, pl.ds(j, L)] = y_v[pl.ds((base - sub) + j, L)]

    it0 = wid * BANDS_PER_W

    def emit(band, it, sem):
        fill(band, it)
        pltpu.async_copy(band, out_hbm.at[pl.ds(8 * it, 8), :], sem)

    def drain(band, sem):
        pltpu.make_async_copy(band, out_hbm.at[pl.ds(0, 8), :], sem).wait()

    emit(band0, it0, sem0)

    def bands(k, _):
        it = it0 + 2 * k + 1
        emit(band1, it, sem1)
        drain(band0, sem0)
        emit(band0, it + 1, sem0)
        drain(band1, sem1)
        return 0

    lax.fori_loop(0, (BANDS_PER_W - 2) // 2, bands, 0)

    emit(band1, it0 + BANDS_PER_W - 1, sem1)
    drain(band0, sem0)
    drain(band1, sem1)


_mesh = plsc.VectorSubcoreMesh(core_axis_name="c", subcore_axis_name="s")

_toeplitz = pl.kernel(
    _body,
    out_type=jax.ShapeDtypeStruct((M, M), jnp.float32),
    mesh=_mesh,
    compiler_params=pltpu.CompilerParams(
        needs_layout_passes=False, use_tc_tiling_on_sc=True
    ),
    scratch_types=[
        pltpu.VMEM((M,), jnp.float32),
        pltpu.VMEM((2 * M,), jnp.float32),
        pltpu.VMEM((8, M), jnp.float32),
        pltpu.VMEM((8, M), jnp.float32),
        pltpu.SemaphoreType.DMA,
        pltpu.SemaphoreType.DMA,
    ],
)


@jax.jit
def kernel(input):
    out = _toeplitz(input.reshape(M).astype(jnp.float32))
    return out[None, :, :]


# speedup vs baseline: 1.0284x; 1.0059x over previous
import jax
import jax.numpy as jnp
from jax import lax
from jax.experimental import pallas as pl
from jax.experimental.pallas import tpu as pltpu
from jax.experimental.pallas import tpu_sc as plsc

M = 4096
NC, NS, L = 2, 16, 16
NW = NC * NS
NBT = M // 8
BANDS_PER_W = NBT // NW


def _body(x_hbm, out_hbm, x_v, y_v, band0, band1, sem0, sem1):
    c = lax.axis_index("c")
    s = lax.axis_index("s")
    wid = s * NC + c

    pltpu.sync_copy(x_hbm, x_v)

    @plsc.parallel_loop(0, M, L)
    def build(tt):
        v = x_v[pl.ds(tt, L)]
        t = tt + lax.iota(jnp.int32, L)
        plsc.store_scatter(y_v, [(M - 1) + t], v)
        plsc.store_scatter(y_v, [(M - 1) - t], v)

    def fill(band, it):
        base = (M - 1) - 8 * it

        @plsc.parallel_loop(0, M, L, unroll=8)
        def chunk(j):
            for sub in range(8):
                band[sub, pl.ds(j, L)] = y_v[pl.ds((base - sub) + j, L)]

    it0 = wid * BANDS_PER_W

    def emit(band, it, sem):
        fill(band, it)
        pltpu.async_copy(band, out_hbm.at[pl.ds(8 * it, 8), :], sem)

    def drain(band, sem):
        pltpu.make_async_copy(band, out_hbm.at[pl.ds(0, 8), :], sem).wait()

    emit(band0, it0, sem0)

    def bands(k, _):
        it = it0 + 2 * k + 1
        emit(band1, it, sem1)
        drain(band0, sem0)
        emit(band0, it + 1, sem0)
        drain(band1, sem1)
        return 0

    lax.fori_loop(0, (BANDS_PER_W - 2) // 2, bands, 0)

    emit(band1, it0 + BANDS_PER_W - 1, sem1)
    drain(band0, sem0)
    drain(band1, sem1)


_mesh = plsc.VectorSubcoreMesh(core_axis_name="c", subcore_axis_name="s")

_toeplitz = pl.kernel(
    _body,
    out_type=jax.ShapeDtypeStruct((M, M), jnp.float32),
    mesh=_mesh,
    compiler_params=pltpu.CompilerParams(
        needs_layout_passes=False, use_tc_tiling_on_sc=True
    ),
    scratch_types=[
        pltpu.VMEM((M,), jnp.float32),
        pltpu.VMEM((2 * M,), jnp.float32),
        pltpu.VMEM((8, M), jnp.float32),
        pltpu.VMEM((8, M), jnp.float32),
        pltpu.SemaphoreType.DMA,
        pltpu.SemaphoreType.DMA,
    ],
)


@jax.jit
def kernel(input):
    out = _toeplitz(input.reshape(M).astype(jnp.float32))
    return out[None, :, :]
